# baseline (speedup 1.0000x reference)
import functools

import jax
import jax.numpy as jnp
from jax import lax
from jax.experimental import pallas as pl
from jax.experimental.pallas import tpu as pltpu
from jax.experimental.pallas import tpu_sc as plsc

BATCH = 128
VOCAB = 100000
THRES = 0.9
R_BLK = 16

_NC = 2
_NS = 16
_NW = _NC * _NS
_RPW = BATCH // _NW
_CH = 20000
_NCH = VOCAB // _CH

_GUMBEL_T = jax.random.gumbel(jax.random.key(42), (BATCH, VOCAB), jnp.float32).T


def _scan_level(hist_ref, nbuck, rank):
    iota = lax.iota(jnp.int32, 16)
    nvec = nbuck // 16

    def body(i, carry):
        cum, found, bkt, res = carry
        j = nvec - 1 - i
        v = hist_ref[pl.ds(j * 16, 16)]
        rv = lax.rev(v, (0,))
        cs = jnp.cumsum(rv)
        tot = jnp.sum(v)
        hit = jnp.logical_and(found == 0, cum + tot >= rank)
        l = jnp.min(jnp.where(cs >= rank - cum, iota, 16))
        above = cum + jnp.sum(jnp.where(iota < l, rv, 0))
        bkt = jnp.where(hit, j * 16 + 15 - l, bkt)
        res = jnp.where(hit, rank - above, res)
        found = jnp.where(hit, 1, found)
        return cum + tot, found, bkt, res

    zero = jnp.int32(0)
    _, _, bkt, res = lax.fori_loop(0, nvec, body, (zero, zero, zero, zero))
    return bkt, res


def _clear(hist_ref, nbuck):
    zeros = jnp.zeros((16,), jnp.int32)

    @plsc.parallel_loop(0, nbuck, 16, unroll=16)
    def body(i):
        hist_ref[pl.ds(i, 16)] = zeros


def _sc_select_threshold(flat_logits):
    k = int((1.0 - THRES) * VOCAB)
    mesh = plsc.VectorSubcoreMesh(core_axis_name="c", subcore_axis_name="s")

    @functools.partial(
        pl.kernel,
        mesh=mesh,
        compiler_params=pltpu.CompilerParams(needs_layout_passes=False),
        out_type=jax.ShapeDtypeStruct((_NW * 16,), jnp.int32),
        scratch_types=[
            pltpu.VMEM((VOCAB,), jnp.float32),
            pltpu.VMEM((4096,), jnp.int32),
            pltpu.VMEM((16,), jnp.int32),
            pltpu.SemaphoreType.DMA,
            pltpu.SemaphoreType.DMA,
        ],
    )
    def sel(x_hbm, out_hbm, row_v, hist_v, res_v, sem0, sem1):
        wid = lax.axis_index("s") * _NC + lax.axis_index("c")
        ones = jnp.ones((16,), jnp.int32)
        iota = lax.iota(jnp.int32, 16)
        sems = (sem0, sem1)

        def row_body(j, res):
            row = wid * _RPW + j

            _clear(hist_v, 4096)
            cps = [None] * _NCH
            cps[0] = pltpu.async_copy(
                x_hbm.at[pl.ds(row * VOCAB, _CH)], row_v.at[pl.ds(0, _CH)],
                sems[0])
            for c in range(_NCH):
                if c + 1 < _NCH:
                    cps[c + 1] = pltpu.async_copy(
                        x_hbm.at[pl.ds(row * VOCAB + (c + 1) * _CH, _CH)],
                        row_v.at[pl.ds((c + 1) * _CH, _CH)],
                        sems[(c + 1) % 2])
                cps[c].wait()

                @plsc.parallel_loop(c * _CH, (c + 1) * _CH, 16, unroll=16)
                def p1(i):
                    sl = pl.ds(i, 16)
                    u = plsc.bitcast(row_v[sl], jnp.int32)
                    um = u ^ ((u >> 31) | jnp.int32(-2147483648))
                    row_v[sl] = plsc.bitcast(um, jnp.float32)
                    idx = lax.shift_right_logical(um, 20)
                    plsc.addupdate_scatter(hist_v, [idx], ones)

            b1, r1 = _scan_level(hist_v, 4096, k)

            _clear(hist_v, 4096)

            @plsc.parallel_loop(0, VOCAB, 16, unroll=16)
            def p2(i):
                sl = pl.ds(i, 16)
                um = plsc.bitcast(row_v[sl], jnp.int32)
                msk = lax.shift_right_logical(um, 20) == b1
                idx = jnp.bitwise_and(lax.shift_right_logical(um, 8),
                                      jnp.int32(0xFFF))
                plsc.addupdate_scatter(hist_v, [idx], ones, mask=msk)

            b2, r2 = _scan_level(hist_v, 4096, r1)

            _clear(hist_v, 256)
            top24 = (b1 << 12) | b2

            @plsc.parallel_loop(0, VOCAB, 16, unroll=16)
            def p3(i):
                sl = pl.ds(i, 16)
                um = plsc.bitcast(row_v[sl], jnp.int32)
                msk = lax.shift_right_logical(um, 8) == top24
                idx = jnp.bitwise_and(um, jnp.int32(0xFF))
                plsc.addupdate_scatter(hist_v, [idx], ones, mask=msk)

            b3, _ = _scan_level(hist_v, 256, r2)

            t = (b1 << 20) | (b2 << 8) | b3
            return jnp.where(iota == j, t, res)

        res_v[...] = lax.fori_loop(0, _RPW, row_body,
                                   jnp.zeros((16,), jnp.int32))
        pltpu.sync_copy(res_v, out_hbm.at[pl.ds(wid * 16, 16)])

    return sel(flat_logits)


_VC = 10000


def _tc_stats_body(x_ref, g_ref, t_ref, m_ref, s_ref, bi_ref,
                   m_run, s_run, by_run, bi_run):
    c = pl.program_id(0)
    nc = pl.num_programs(0)
    x = x_ref[...]
    g = g_ref[...]
    t = t_ref[...]

    @pl.when(c == 0)
    def _():
        m_run[...] = jnp.full(m_run.shape, -jnp.inf, jnp.float32)
        s_run[...] = jnp.zeros(s_run.shape, jnp.float32)
        by_run[...] = jnp.full(by_run.shape, -jnp.inf, jnp.float32)
        bi_run[...] = jnp.zeros(bi_run.shape, jnp.int32)

    keep = x >= t
    m_old = m_run[...]
    m_new = jnp.maximum(m_old, jnp.max(x, axis=0, keepdims=True))
    e = jnp.where(keep, jnp.exp(x - m_new), 0.0)
    s_new = s_run[...] * jnp.exp(m_old - m_new) + jnp.sum(e, axis=0,
                                                          keepdims=True)
    m_run[...] = m_new
    s_run[...] = s_new

    y = jnp.where(keep, x + g, -jnp.inf)
    cy = jnp.max(y, axis=0, keepdims=True)
    iota = jax.lax.broadcasted_iota(jnp.int32, y.shape, 0) + c * x.shape[0]
    ci = jnp.min(jnp.where(y == cy, iota, VOCAB), axis=0, keepdims=True)
    better = cy > by_run[...]
    by_run[...] = jnp.where(better, cy, by_run[...])
    bi_run[...] = jnp.where(better, ci, bi_run[...])

    @pl.when(c == nc - 1)
    def _():
        m_ref[...] = m_run[...]
        s_ref[...] = s_run[...]
        bi_ref[...] = bi_run[...]


def _tc_probs_body(x_ref, t_ref, m_ref, s_ref, probs_ref):
    x = x_ref[...]
    keep = x >= t_ref[...]
    e = jnp.where(keep, jnp.exp(x - m_ref[...]), 0.0)
    probs_ref[...] = e / s_ref[...]


def kernel(logits):
    b, v = logits.shape
    flat = lax.reshape(logits.T, (b * v,), dimensions=(1, 0))
    thr = _sc_select_threshold(flat).reshape(_NW, 16)
    tk = thr[:, :_RPW].reshape(1, b)
    tu = jnp.where(tk < 0, tk & jnp.int32(0x7FFFFFFF), ~tk)
    t = jax.lax.bitcast_convert_type(tu, jnp.float32)

    x_t = logits.T
    g_t = _GUMBEL_T
    grid = (v // _VC,)
    chunk_spec = pl.BlockSpec((_VC, b), lambda c: (c, 0))
    scal_spec = pl.BlockSpec((1, b), lambda c: (0, 0))
    scal = jax.ShapeDtypeStruct((1, b), jnp.float32)
    m, s, bi = pl.pallas_call(
        _tc_stats_body,
        grid=grid,
        in_specs=[chunk_spec, chunk_spec, scal_spec],
        out_specs=[scal_spec, scal_spec, scal_spec],
        out_shape=[scal, scal, jax.ShapeDtypeStruct((1, b), jnp.int32)],
        scratch_shapes=[pltpu.VMEM((1, b), jnp.float32),
                        pltpu.VMEM((1, b), jnp.float32),
                        pltpu.VMEM((1, b), jnp.float32),
                        pltpu.VMEM((1, b), jnp.int32)],
    )(x_t, g_t, t)
    probs_t = pl.pallas_call(
        _tc_probs_body,
        grid=grid,
        in_specs=[chunk_spec, scal_spec, scal_spec, scal_spec],
        out_specs=chunk_spec,
        out_shape=jax.ShapeDtypeStruct((v, b), jnp.float32),
    )(x_t, t, m, s)
    return bi.reshape(b, 1), probs_t.T


# speedup vs baseline: 1.2634x; 1.2634x over previous
import functools

import jax
import jax.numpy as jnp
from jax import lax
from jax.experimental import pallas as pl
from jax.experimental.pallas import tpu as pltpu
from jax.experimental.pallas import tpu_sc as plsc

BATCH = 128
VOCAB = 100000
THRES = 0.9
R_BLK = 16

_NC = 2
_NS = 16
_NW = _NC * _NS
_RPW = BATCH // _NW
_CH = 10000
_NCH = VOCAB // _CH
_CPAD = 10112

_GUMBEL_T = jax.random.gumbel(jax.random.key(42), (BATCH, VOCAB), jnp.float32).T


def _scan_level(hist_ref, nbuck, rank):
    iota = lax.iota(jnp.int32, 16)
    nvec = nbuck // 16

    def body(i, carry):
        cum, found, bkt, res = carry
        j = nvec - 1 - i
        v = hist_ref[pl.ds(j * 16, 16)]
        rv = lax.rev(v, (0,))
        cs = jnp.cumsum(rv)
        tot = jnp.sum(v)
        hit = jnp.logical_and(found == 0, cum + tot >= rank)
        l = jnp.min(jnp.where(cs >= rank - cum, iota, 16))
        above = cum + jnp.sum(jnp.where(iota < l, rv, 0))
        bkt = jnp.where(hit, j * 16 + 15 - l, bkt)
        res = jnp.where(hit, rank - above, res)
        found = jnp.where(hit, 1, found)
        return cum + tot, found, bkt, res

    zero = jnp.int32(0)
    _, _, bkt, res = lax.fori_loop(0, nvec, body, (zero, zero, zero, zero))
    return bkt, res


def _clear(hist_ref, nbuck):
    zeros = jnp.zeros((16,), jnp.int32)

    @plsc.parallel_loop(0, nbuck, 16, unroll=16)
    def body(i):
        hist_ref[pl.ds(i, 16)] = zeros


def _sc_select_threshold(flat_logits):
    k = int((1.0 - THRES) * VOCAB)
    mesh = plsc.VectorSubcoreMesh(core_axis_name="c", subcore_axis_name="s")

    @functools.partial(
        pl.kernel,
        mesh=mesh,
        compiler_params=pltpu.CompilerParams(needs_layout_passes=False),
        out_type=jax.ShapeDtypeStruct((_NW * 16,), jnp.int32),
        scratch_types=[
            pltpu.VMEM((VOCAB,), jnp.float32),
            pltpu.VMEM((4096,), jnp.int32),
            pltpu.VMEM((16,), jnp.int32),
            pltpu.SemaphoreType.DMA,
            pltpu.SemaphoreType.DMA,
        ],
    )
    def sel(x_hbm, out_hbm, row_v, hist_v, res_v, sem0, sem1):
        wid = lax.axis_index("s") * _NC + lax.axis_index("c")
        ones = jnp.ones((16,), jnp.int32)
        iota = lax.iota(jnp.int32, 16)
        sems = (sem0, sem1)

        def row_body(j, res):
            row = wid * _RPW + j

            _clear(hist_v, 4096)
            cps = [None] * _NCH
            cps[0] = pltpu.async_copy(
                x_hbm.at[pl.ds(row * _CPAD, _CH)], row_v.at[pl.ds(0, _CH)],
                sems[0])
            for c in range(_NCH):
                if c + 1 < _NCH:
                    cps[c + 1] = pltpu.async_copy(
                        x_hbm.at[pl.ds(((c + 1) * BATCH + row) * _CPAD, _CH)],
                        row_v.at[pl.ds((c + 1) * _CH, _CH)],
                        sems[(c + 1) % 2])
                cps[c].wait()

                @plsc.parallel_loop(c * _CH, (c + 1) * _CH, 16, unroll=16)
                def p1(i):
                    sl = pl.ds(i, 16)
                    u = plsc.bitcast(row_v[sl], jnp.int32)
                    um = u ^ ((u >> 31) | jnp.int32(-2147483648))
                    row_v[sl] = plsc.bitcast(um, jnp.float32)
                    idx = lax.shift_right_logical(um, 20)
                    plsc.addupdate_scatter(hist_v, [idx], ones)

            b1, r1 = _scan_level(hist_v, 4096, k)

            _clear(hist_v, 4096)

            @plsc.parallel_loop(0, VOCAB, 16, unroll=16)
            def p2(i):
                sl = pl.ds(i, 16)
                um = plsc.bitcast(row_v[sl], jnp.int32)
                msk = lax.shift_right_logical(um, 20) == b1
                idx = jnp.bitwise_and(lax.shift_right_logical(um, 8),
                                      jnp.int32(0xFFF))
                plsc.addupdate_scatter(hist_v, [idx], ones, mask=msk)

            b2, r2 = _scan_level(hist_v, 4096, r1)

            _clear(hist_v, 256)
            top24 = (b1 << 12) | b2

            @plsc.parallel_loop(0, VOCAB, 16, unroll=16)
            def p3(i):
                sl = pl.ds(i, 16)
                um = plsc.bitcast(row_v[sl], jnp.int32)
                msk = lax.shift_right_logical(um, 8) == top24
                idx = jnp.bitwise_and(um, jnp.int32(0xFF))
                plsc.addupdate_scatter(hist_v, [idx], ones, mask=msk)

            b3, _ = _scan_level(hist_v, 256, r2)

            t = (b1 << 20) | (b2 << 8) | b3
            return jnp.where(iota == j, t, res)

        res_v[...] = lax.fori_loop(0, _RPW, row_body,
                                   jnp.zeros((16,), jnp.int32))
        pltpu.sync_copy(res_v, out_hbm.at[pl.ds(wid * 16, 16)])

    return sel(flat_logits)


_VC = 10000


def _tc_relayout_body(x_ref, out_ref, buf, sem):
    c = pl.program_id(0)
    buf[:, : _VC] = x_ref[...].T
    cps = [pltpu.make_async_copy(
        buf.at[r], out_ref.at[pl.ds((c * BATCH + r) * _CPAD, _CPAD)], sem)
        for r in range(BATCH)]
    for cp in cps:
        cp.start()
    for cp in cps:
        cp.wait()


def _tc_stats_body(x_ref, g_ref, t_ref, m_ref, s_ref, bi_ref,
                   m_run, s_run, by_run, bi_run):
    c = pl.program_id(0)
    nc = pl.num_programs(0)
    x = x_ref[...]
    g = g_ref[...]
    t = t_ref[...]

    @pl.when(c == 0)
    def _():
        m_run[...] = jnp.full(m_run.shape, -jnp.inf, jnp.float32)
        s_run[...] = jnp.zeros(s_run.shape, jnp.float32)
        by_run[...] = jnp.full(by_run.shape, -jnp.inf, jnp.float32)
        bi_run[...] = jnp.zeros(bi_run.shape, jnp.int32)

    keep = x >= t
    m_old = m_run[...]
    m_new = jnp.maximum(m_old, jnp.max(x, axis=0, keepdims=True))
    e = jnp.where(keep, jnp.exp(x - m_new), 0.0)
    s_new = s_run[...] * jnp.exp(m_old - m_new) + jnp.sum(e, axis=0,
                                                          keepdims=True)
    m_run[...] = m_new
    s_run[...] = s_new

    y = jnp.where(keep, x + g, -jnp.inf)
    cy = jnp.max(y, axis=0, keepdims=True)
    iota = jax.lax.broadcasted_iota(jnp.int32, y.shape, 0) + c * x.shape[0]
    ci = jnp.min(jnp.where(y == cy, iota, VOCAB), axis=0, keepdims=True)
    better = cy > by_run[...]
    by_run[...] = jnp.where(better, cy, by_run[...])
    bi_run[...] = jnp.where(better, ci, bi_run[...])

    @pl.when(c == nc - 1)
    def _():
        m_ref[...] = m_run[...]
        s_ref[...] = s_run[...]
        bi_ref[...] = bi_run[...]


def _tc_probs_body(x_ref, t_ref, m_ref, s_ref, probs_ref):
    x = x_ref[...]
    keep = x >= t_ref[...]
    e = jnp.where(keep, jnp.exp(x - m_ref[...]), 0.0)
    probs_ref[...] = e / s_ref[...]


def kernel(logits):
    b, v = logits.shape
    x_t0 = logits.T
    flat = pl.pallas_call(
        _tc_relayout_body,
        grid=(v // _VC,),
        in_specs=[pl.BlockSpec((_VC, b), lambda c: (c, 0))],
        out_specs=pl.BlockSpec(memory_space=pltpu.MemorySpace.HBM),
        out_shape=jax.ShapeDtypeStruct((_NCH * b * _CPAD,), jnp.float32),
        scratch_shapes=[pltpu.VMEM((b, _CPAD), jnp.float32),
                        pltpu.SemaphoreType.DMA],
    )(x_t0)
    thr = _sc_select_threshold(flat).reshape(_NW, 16)
    tk = thr[:, :_RPW].reshape(1, b)
    tu = jnp.where(tk < 0, tk & jnp.int32(0x7FFFFFFF), ~tk)
    t = jax.lax.bitcast_convert_type(tu, jnp.float32)

    x_t = logits.T
    g_t = _GUMBEL_T
    grid = (v // _VC,)
    chunk_spec = pl.BlockSpec((_VC, b), lambda c: (c, 0))
    scal_spec = pl.BlockSpec((1, b), lambda c: (0, 0))
    scal = jax.ShapeDtypeStruct((1, b), jnp.float32)
    m, s, bi = pl.pallas_call(
        _tc_stats_body,
        grid=grid,
        in_specs=[chunk_spec, chunk_spec, scal_spec],
        out_specs=[scal_spec, scal_spec, scal_spec],
        out_shape=[scal, scal, jax.ShapeDtypeStruct((1, b), jnp.int32)],
        scratch_shapes=[pltpu.VMEM((1, b), jnp.float32),
                        pltpu.VMEM((1, b), jnp.float32),
                        pltpu.VMEM((1, b), jnp.float32),
                        pltpu.VMEM((1, b), jnp.int32)],
    )(x_t, g_t, t)
    probs_t = pl.pallas_call(
        _tc_probs_body,
        grid=grid,
        in_specs=[chunk_spec, scal_spec, scal_spec, scal_spec],
        out_specs=chunk_spec,
        out_shape=jax.ShapeDtypeStruct((v, b), jnp.float32),
    )(x_t, t, m, s)
    return bi.reshape(b, 1), probs_t.T


# speedup vs baseline: 1.3192x; 1.0442x over previous
import functools

import jax
import jax.numpy as jnp
from jax import lax
from jax.experimental import pallas as pl
from jax.experimental.pallas import tpu as pltpu
from jax.experimental.pallas import tpu_sc as plsc

BATCH = 128
VOCAB = 100000
THRES = 0.9
R_BLK = 16

_NC = 2
_NS = 16
_NW = _NC * _NS
_RPW = BATCH // _NW
_CH = 10000
_NCH = VOCAB // _CH
_CPAD = 10112

_GUMBEL_T = jax.random.gumbel(jax.random.key(42), (BATCH, VOCAB), jnp.float32).T


def _scan_level(hist_ref, nbuck, rank):
    nvec = nbuck // 16

    def light(i, carry):
        cum, found, jv, cumv = carry
        j = nvec - 1 - i
        tot = jnp.sum(hist_ref[pl.ds(j * 16, 16)])
        hit = jnp.logical_and(found == 0, cum + tot >= rank)
        jv = jnp.where(hit, j, jv)
        cumv = jnp.where(hit, cum, cumv)
        found = jnp.where(hit, 1, found)
        return cum + tot, found, jv, cumv

    zero = jnp.int32(0)
    _, _, jv, cumv = lax.fori_loop(0, nvec, light, (zero, zero, zero, zero))
    rv = lax.rev(hist_ref[pl.ds(jv * 16, 16)], (0,))
    cs = jnp.cumsum(rv)
    iota = lax.iota(jnp.int32, 16)
    l = jnp.min(jnp.where(cs >= rank - cumv, iota, 16))
    above = cumv + jnp.sum(jnp.where(iota < l, rv, 0))
    return jv * 16 + 15 - l, rank - above


def _clear(hist_ref, nbuck):
    zeros = jnp.zeros((16,), jnp.int32)

    @plsc.parallel_loop(0, nbuck, 16, unroll=16)
    def body(i):
        hist_ref[pl.ds(i, 16)] = zeros


def _sc_select_threshold(flat_logits):
    k = int((1.0 - THRES) * VOCAB)
    mesh = plsc.VectorSubcoreMesh(core_axis_name="c", subcore_axis_name="s")

    @functools.partial(
        pl.kernel,
        mesh=mesh,
        compiler_params=pltpu.CompilerParams(needs_layout_passes=False),
        out_type=jax.ShapeDtypeStruct((_NW * 16,), jnp.int32),
        scratch_types=[
            pltpu.VMEM((VOCAB,), jnp.float32),
            pltpu.VMEM((4096,), jnp.int32),
            pltpu.VMEM((16,), jnp.int32),
            pltpu.SemaphoreType.DMA,
            pltpu.SemaphoreType.DMA,
        ],
    )
    def sel(x_hbm, out_hbm, row_v, hist_v, res_v, sem0, sem1):
        wid = lax.axis_index("s") * _NC + lax.axis_index("c")
        ones = jnp.ones((16,), jnp.int32)
        iota = lax.iota(jnp.int32, 16)
        sems = (sem0, sem1)

        def row_body(j, res):
            row = wid * _RPW + j

            _clear(hist_v, 2048)
            cps = [None] * _NCH
            cps[0] = pltpu.async_copy(
                x_hbm.at[pl.ds(row * _CPAD, _CH)], row_v.at[pl.ds(0, _CH)],
                sems[0])
            for c in range(_NCH):
                if c + 1 < _NCH:
                    cps[c + 1] = pltpu.async_copy(
                        x_hbm.at[pl.ds(((c + 1) * BATCH + row) * _CPAD, _CH)],
                        row_v.at[pl.ds((c + 1) * _CH, _CH)],
                        sems[(c + 1) % 2])
                cps[c].wait()

                @plsc.parallel_loop(c * _CH, (c + 1) * _CH, 16, unroll=16)
                def p1(i):
                    sl = pl.ds(i, 16)
                    u = plsc.bitcast(row_v[sl], jnp.int32)
                    um = u ^ ((u >> 31) | jnp.int32(-2147483648))
                    row_v[sl] = plsc.bitcast(um, jnp.float32)
                    idx = lax.shift_right_logical(um, 21)
                    plsc.addupdate_scatter(hist_v, [idx], ones)

            b1, r1 = _scan_level(hist_v, 2048, k)

            _clear(hist_v, 2048)

            @plsc.parallel_loop(0, VOCAB, 16, unroll=16)
            def p2(i):
                sl = pl.ds(i, 16)
                um = plsc.bitcast(row_v[sl], jnp.int32)
                msk = lax.shift_right_logical(um, 21) == b1
                idx = jnp.bitwise_and(lax.shift_right_logical(um, 10),
                                      jnp.int32(0x7FF))
                plsc.addupdate_scatter(hist_v, [idx], ones, mask=msk)

            b2, r2 = _scan_level(hist_v, 2048, r1)

            _clear(hist_v, 1024)
            top22 = (b1 << 11) | b2

            @plsc.parallel_loop(0, VOCAB, 16, unroll=16)
            def p3(i):
                sl = pl.ds(i, 16)
                um = plsc.bitcast(row_v[sl], jnp.int32)
                msk = lax.shift_right_logical(um, 10) == top22
                idx = jnp.bitwise_and(um, jnp.int32(0x3FF))
                plsc.addupdate_scatter(hist_v, [idx], ones, mask=msk)

            b3, _ = _scan_level(hist_v, 1024, r2)

            t = (b1 << 21) | (b2 << 10) | b3
            return jnp.where(iota == j, t, res)

        res_v[...] = lax.fori_loop(0, _RPW, row_body,
                                   jnp.zeros((16,), jnp.int32))
        pltpu.sync_copy(res_v, out_hbm.at[pl.ds(wid * 16, 16)])

    return sel(flat_logits)


_VC = 10000


def _tc_relayout_body(x_ref, out_ref, buf, sem):
    c = pl.program_id(0)
    buf[:, : _VC] = x_ref[...].T
    cps = [pltpu.make_async_copy(
        buf.at[r], out_ref.at[pl.ds((c * BATCH + r) * _CPAD, _CPAD)], sem)
        for r in range(BATCH)]
    for cp in cps:
        cp.start()
    for cp in cps:
        cp.wait()


def _tc_stats_body(x_ref, g_ref, t_ref, m_ref, s_ref, bi_ref,
                   m_run, s_run, by_run, bi_run):
    c = pl.program_id(0)
    nc = pl.num_programs(0)
    x = x_ref[...]
    g = g_ref[...]
    t = t_ref[...]

    @pl.when(c == 0)
    def _():
        m_run[...] = jnp.full(m_run.shape, -jnp.inf, jnp.float32)
        s_run[...] = jnp.zeros(s_run.shape, jnp.float32)
        by_run[...] = jnp.full(by_run.shape, -jnp.inf, jnp.float32)
        bi_run[...] = jnp.zeros(bi_run.shape, jnp.int32)

    keep = x >= t
    m_old = m_run[...]
    m_new = jnp.maximum(m_old, jnp.max(x, axis=0, keepdims=True))
    e = jnp.where(keep, jnp.exp(x - m_new), 0.0)
    s_new = s_run[...] * jnp.exp(m_old - m_new) + jnp.sum(e, axis=0,
                                                          keepdims=True)
    m_run[...] = m_new
    s_run[...] = s_new

    y = jnp.where(keep, x + g, -jnp.inf)
    cy = jnp.max(y, axis=0, keepdims=True)
    iota = jax.lax.broadcasted_iota(jnp.int32, y.shape, 0) + c * x.shape[0]
    ci = jnp.min(jnp.where(y == cy, iota, VOCAB), axis=0, keepdims=True)
    better = cy > by_run[...]
    by_run[...] = jnp.where(better, cy, by_run[...])
    bi_run[...] = jnp.where(better, ci, bi_run[...])

    @pl.when(c == nc - 1)
    def _():
        m_ref[...] = m_run[...]
        s_ref[...] = s_run[...]
        bi_ref[...] = bi_run[...]


def _tc_probs_body(x_ref, t_ref, m_ref, s_ref, probs_ref):
    x = x_ref[...]
    keep = x >= t_ref[...]
    e = jnp.where(keep, jnp.exp(x - m_ref[...]), 0.0)
    probs_ref[...] = e / s_ref[...]


def kernel(logits):
    b, v = logits.shape
    x_t0 = logits.T
    flat = pl.pallas_call(
        _tc_relayout_body,
        grid=(v // _VC,),
        in_specs=[pl.BlockSpec((_VC, b), lambda c: (c, 0))],
        out_specs=pl.BlockSpec(memory_space=pltpu.MemorySpace.HBM),
        out_shape=jax.ShapeDtypeStruct((_NCH * b * _CPAD,), jnp.float32),
        scratch_shapes=[pltpu.VMEM((b, _CPAD), jnp.float32),
                        pltpu.SemaphoreType.DMA],
    )(x_t0)
    thr = _sc_select_threshold(flat).reshape(_NW, 16)
    tk = thr[:, :_RPW].reshape(1, b)
    tu = jnp.where(tk < 0, tk & jnp.int32(0x7FFFFFFF), ~tk)
    t = jax.lax.bitcast_convert_type(tu, jnp.float32)

    x_t = logits.T
    g_t = _GUMBEL_T
    grid = (v // _VC,)
    chunk_spec = pl.BlockSpec((_VC, b), lambda c: (c, 0))
    scal_spec = pl.BlockSpec((1, b), lambda c: (0, 0))
    scal = jax.ShapeDtypeStruct((1, b), jnp.float32)
    m, s, bi = pl.pallas_call(
        _tc_stats_body,
        grid=grid,
        in_specs=[chunk_spec, chunk_spec, scal_spec],
        out_specs=[scal_spec, scal_spec, scal_spec],
        out_shape=[scal, scal, jax.ShapeDtypeStruct((1, b), jnp.int32)],
        scratch_shapes=[pltpu.VMEM((1, b), jnp.float32),
                        pltpu.VMEM((1, b), jnp.float32),
                        pltpu.VMEM((1, b), jnp.float32),
                        pltpu.VMEM((1, b), jnp.int32)],
    )(x_t, g_t, t)
    probs_t = pl.pallas_call(
        _tc_probs_body,
        grid=grid,
        in_specs=[chunk_spec, scal_spec, scal_spec, scal_spec],
        out_specs=chunk_spec,
        out_shape=jax.ShapeDtypeStruct((v, b), jnp.float32),
    )(x_t, t, m, s)
    return bi.reshape(b, 1), probs_t.T
